# baseline (speedup 1.0000x reference)
import dataclasses
import functools

import jax
import jax.numpy as jnp
from jax import lax
from jax.experimental import pallas as pl
from jax.experimental.pallas import tpu as pltpu
from jax.experimental.pallas import tpu_sc as plsc

N = 10000
D = 128
NUM_LAYERS = 4
E2 = 640000

NPAD = 10240
HALF = 5120
HROWS = 5128
NW = 32
ACC_RPT = 320

EPT = 20096
E2PAD = NW * EPT
PAD_IDX = 10239
DUMP = HALF

CAP = 5504
LCH = CAP // 128
CHUNK = 128
NBUF = 2


def _mm_body(h_ref, w0_ref, w1_ref, b_ref, self_ref, neigh_ref):
    h = h_ref[...]
    dn = (((1,), (1,)), ((), ()))
    self_ref[...] = lax.dot_general(
        h, w0_ref[...], dn, preferred_element_type=jnp.float32) + b_ref[...]
    neigh_ref[...] = lax.dot_general(
        h, w1_ref[...], dn, preferred_element_type=jnp.float32)


_tc_mm = pl.pallas_call(
    _mm_body,
    out_shape=(jax.ShapeDtypeStruct((NPAD, D), jnp.float32),
               jax.ShapeDtypeStruct((NPAD, D), jnp.float32)),
)


def _combine_body(relu, self_ref, p_ref, o_ref):
    lo = p_ref[0, :HALF]
    hi = p_ref[1, :HALF]
    v = self_ref[...] + jnp.concatenate([lo, hi], axis=0)
    rows = lax.broadcasted_iota(jnp.int32, (NPAD, D), 0)
    v = jnp.where(rows < N, v, 0.0)
    if relu:
        v = jnp.maximum(v, 0.0)
    o_ref[...] = v


def _tc_combine(selfp, parts, relu):
    return pl.pallas_call(
        functools.partial(_combine_body, relu),
        out_shape=jax.ShapeDtypeStruct((NPAD, D), jnp.float32),
    )(selfp, parts)


_sc_mesh = plsc.VectorSubcoreMesh(core_axis_name="c", subcore_axis_name="s")

_part_cp = pltpu.CompilerParams()
if "needs_layout_passes" in pltpu.CompilerParams.__dataclass_fields__:
    _part_cp = dataclasses.replace(_part_cp, needs_layout_passes=False)


@functools.partial(
    pl.kernel,
    out_type=(jax.ShapeDtypeStruct((NW * 4 * CAP,), jnp.int32),
              jax.ShapeDtypeStruct((NW * 4 * CAP,), jnp.int32)),
    mesh=_sc_mesh,
    scratch_types=[
        pltpu.VMEM((EPT // 128, 128), jnp.int32),
        pltpu.VMEM((EPT // 128, 128), jnp.int32),
        pltpu.VMEM((4 * CAP + 16,), jnp.int32),
        pltpu.VMEM((4 * CAP + 16,), jnp.int32),
    ],
    compiler_params=_part_cp,
)
def _sc_part(srcr_hbm, dstr_hbm, srcl_hbm, dstl_hbm,
             src_in, dst_in, srcb, dstb):
    cid = lax.axis_index("c")
    sid = lax.axis_index("s")
    wid = cid * 16 + sid
    pltpu.sync_copy(srcr_hbm.at[wid], src_in)
    pltpu.sync_copy(dstr_hbm.at[wid], dst_in)

    dump = jnp.full((16,), DUMP, jnp.int32)

    @pl.loop(0, 4 * CAP // 16)
    def _(i):
        srcb[pl.ds(i * 16, 16)] = dump
        dstb[pl.ds(i * 16, 16)] = dump

    def body(j, pos):
        for k in range(8):
            s16 = src_in[j, pl.ds(k * 16, 16)]
            d16 = dst_in[j, pl.ds(k * 16, 16)]
            gs = s16 >= HALF
            gd = d16 >= HALF
            valid = s16 != PAD_IDX
            sm = s16 - jnp.where(gs, HALF, 0)
            dm = d16 - jnp.where(gd, HALF, 0)
            for g in range(4):
                mg = (gs == (g >= 2)) & (gd == (g % 2 == 1)) & valid
                mi = jnp.where(mg, 1, 0)
                ranks = plsc.cumsum(mi)
                lidx = jnp.where(mg, g * CAP + pos[g] + ranks - 1, 4 * CAP)
                plsc.store_scatter(srcb, [lidx], sm)
                plsc.store_scatter(dstb, [lidx], dm)
                cnt = jnp.sum(mi)
                pos = tuple(p + cnt if gg == g else p
                            for gg, p in enumerate(pos))
        return pos

    lax.fori_loop(0, EPT // 128, body,
                  (jnp.int32(0), jnp.int32(0), jnp.int32(0), jnp.int32(0)))

    for g in range(4):
        pltpu.sync_copy(srcb.at[pl.ds(g * CAP, CAP)],
                        srcl_hbm.at[pl.ds((wid * 4 + g) * CAP, CAP)])
        pltpu.sync_copy(dstb.at[pl.ds(g * CAP, CAP)],
                        dstl_hbm.at[pl.ds((wid * 4 + g) * CAP, CAP)])


@functools.partial(
    pl.kernel,
    out_type=jax.ShapeDtypeStruct((2, HROWS, D), jnp.float32),
    mesh=_sc_mesh,
    scratch_types=[
        pltpu.VMEM((LCH, CHUNK), jnp.int32),
        pltpu.VMEM((LCH, CHUNK), jnp.int32),
    ] + [pltpu.VMEM((CHUNK, D), jnp.float32)] * NBUF
      + [pltpu.VMEM_SHARED((HROWS, D), jnp.float32),
         pltpu.VMEM_SHARED((HROWS, D), jnp.float32)]
      + [pltpu.SemaphoreType.DMA] * NBUF,
)
def _sc_agg(neigh_hbm, srcl_hbm, dstl_hbm, zeros_hbm, out_hbm,
            src_v, dst_v, *rest):
    bufs = rest[:NBUF]
    tab_sh = rest[NBUF]
    acc_sh = rest[NBUF + 1]
    sems = rest[NBUF + 2:]
    cid = lax.axis_index("c")
    sid = lax.axis_index("s")
    row0 = sid * ACC_RPT
    pltpu.sync_copy(zeros_hbm.at[pl.ds(row0, ACC_RPT)],
                    acc_sh.at[pl.ds(row0, ACC_RPT)])

    @pl.when(sid == 15)
    def _():
        pltpu.sync_copy(zeros_hbm.at[pl.ds(HALF, 8)],
                        acc_sh.at[pl.ds(HALF, 8)])

    def g_start(j, k):
        pltpu.async_copy(tab_sh.at[src_v.at[j]], bufs[k], sems[k])

    def g_wait(k):
        pltpu.make_async_copy(tab_sh.at[src_v.at[0]], bufs[k], sems[k]).wait()

    def s_add(j, k):
        pltpu.sync_copy(bufs[k], acc_sh.at[dst_v.at[j]], add=True)

    for p in range(2):
        pltpu.sync_copy(neigh_hbm.at[pl.ds(p * HALF + row0, ACC_RPT)],
                        tab_sh.at[pl.ds(row0, ACC_RPT)])

        @pl.when(sid == 15)
        def _():
            pltpu.sync_copy(zeros_hbm.at[pl.ds(HALF, 8)],
                            tab_sh.at[pl.ds(HALF, 8)])

        plsc.subcore_barrier()

        g = p * 2 + cid
        for oo in range(2):
            o = sid * 2 + oo
            pltpu.sync_copy(srcl_hbm.at[o, g], src_v)
            pltpu.sync_copy(dstl_hbm.at[o, g], dst_v)
            for k in range(NBUF - 1):
                g_start(k, k)

            @pl.loop(0, LCH, step=NBUF)
            def _(j):
                for k in range(NBUF):
                    nxt = j + k + NBUF - 1

                    @pl.when(nxt < LCH)
                    def _():
                        g_start(nxt, (k + NBUF - 1) % NBUF)

                    @pl.when(j + k < LCH)
                    def _():
                        g_wait(k)
                        s_add(j + k, k)

        plsc.subcore_barrier()

    pltpu.sync_copy(acc_sh.at[pl.ds(row0, ACC_RPT)],
                    out_hbm.at[cid, pl.ds(row0, ACC_RPT)])


def kernel(x, edges, W0, W1, b):
    e = edges.astype(jnp.int32)
    src = jnp.concatenate([e[:, 0], e[:, 1]])
    dst = jnp.concatenate([e[:, 1], e[:, 0]])
    pad = jnp.full((E2PAD - E2,), PAD_IDX, jnp.int32)
    srcr = jnp.concatenate([src, pad]).reshape(NW, EPT // 128, 128)
    dstr = jnp.concatenate([dst, pad]).reshape(NW, EPT // 128, 128)
    zeros = jnp.zeros((HROWS, D), jnp.float32)

    srcl, dstl = _sc_part(srcr, dstr)
    srcl = srcl.reshape(NW, 4, LCH, 128)
    dstl = dstl.reshape(NW, 4, LCH, 128)

    h = jnp.zeros((NPAD, D), jnp.float32).at[:N].set(x)
    for l in range(NUM_LAYERS):
        selfp, neigh = _tc_mm(h, W0[l], W1[l], b[l].reshape(1, D))
        parts = _sc_agg(neigh, srcl, dstl, zeros)
        h = _tc_combine(selfp, parts, relu=(l < NUM_LAYERS - 1))
    return h[:N]


# speedup vs baseline: 1.0124x; 1.0124x over previous
import dataclasses
import functools

import jax
import jax.numpy as jnp
from jax import lax
from jax.experimental import pallas as pl
from jax.experimental.pallas import tpu as pltpu
from jax.experimental.pallas import tpu_sc as plsc

N = 10000
D = 128
NUM_LAYERS = 4
E2 = 640000

NPAD = 10240
HALF = 5120
HROWS = 5128
NW = 32
ACC_RPT = 320

EPT = 20096
E2PAD = NW * EPT
PAD_IDX = 10239
DUMP = HALF

CAP = 5504
LCH = CAP // 128
CHUNK = 128
NBUF = 2


def _mm_body(h_ref, w0_ref, w1_ref, b_ref, self_ref, neigh_ref):
    h = h_ref[...]
    dn = (((1,), (1,)), ((), ()))
    self_ref[...] = lax.dot_general(
        h, w0_ref[...], dn, preferred_element_type=jnp.float32) + b_ref[...]
    neigh_ref[...] = lax.dot_general(
        h, w1_ref[...], dn, preferred_element_type=jnp.float32)


_tc_mm = pl.pallas_call(
    _mm_body,
    out_shape=(jax.ShapeDtypeStruct((NPAD, D), jnp.float32),
               jax.ShapeDtypeStruct((NPAD, D), jnp.float32)),
)


def _mm_fused_body(self_ref, p_ref, w0_ref, w1_ref, b_ref,
                   self_out, neigh_out):
    lo = p_ref[0, :HALF]
    hi = p_ref[1, :HALF]
    v = self_ref[...] + jnp.concatenate([lo, hi], axis=0)
    rows = lax.broadcasted_iota(jnp.int32, (NPAD, D), 0)
    h = jnp.maximum(jnp.where(rows < N, v, 0.0), 0.0)
    dn = (((1,), (1,)), ((), ()))
    self_out[...] = lax.dot_general(
        h, w0_ref[...], dn, preferred_element_type=jnp.float32) + b_ref[...]
    neigh_out[...] = lax.dot_general(
        h, w1_ref[...], dn, preferred_element_type=jnp.float32)


_tc_mm_fused = pl.pallas_call(
    _mm_fused_body,
    out_shape=(jax.ShapeDtypeStruct((NPAD, D), jnp.float32),
               jax.ShapeDtypeStruct((NPAD, D), jnp.float32)),
)


def _combine_body(relu, self_ref, p_ref, o_ref):
    lo = p_ref[0, :HALF]
    hi = p_ref[1, :HALF]
    v = self_ref[...] + jnp.concatenate([lo, hi], axis=0)
    rows = lax.broadcasted_iota(jnp.int32, (NPAD, D), 0)
    v = jnp.where(rows < N, v, 0.0)
    if relu:
        v = jnp.maximum(v, 0.0)
    o_ref[...] = v


def _tc_combine(selfp, parts, relu):
    return pl.pallas_call(
        functools.partial(_combine_body, relu),
        out_shape=jax.ShapeDtypeStruct((NPAD, D), jnp.float32),
    )(selfp, parts)


_sc_mesh = plsc.VectorSubcoreMesh(core_axis_name="c", subcore_axis_name="s")

_part_cp = pltpu.CompilerParams()
if "needs_layout_passes" in pltpu.CompilerParams.__dataclass_fields__:
    _part_cp = dataclasses.replace(_part_cp, needs_layout_passes=False)


@functools.partial(
    pl.kernel,
    out_type=(jax.ShapeDtypeStruct((NW * 4 * CAP,), jnp.int32),
              jax.ShapeDtypeStruct((NW * 4 * CAP,), jnp.int32)),
    mesh=_sc_mesh,
    scratch_types=[
        pltpu.VMEM((EPT // 128, 128), jnp.int32),
        pltpu.VMEM((EPT // 128, 128), jnp.int32),
        pltpu.VMEM((4 * CAP + 16,), jnp.int32),
        pltpu.VMEM((4 * CAP + 16,), jnp.int32),
    ],
    compiler_params=_part_cp,
)
def _sc_part(srcr_hbm, dstr_hbm, srcl_hbm, dstl_hbm,
             src_in, dst_in, srcb, dstb):
    cid = lax.axis_index("c")
    sid = lax.axis_index("s")
    wid = cid * 16 + sid
    pltpu.sync_copy(srcr_hbm.at[wid], src_in)
    pltpu.sync_copy(dstr_hbm.at[wid], dst_in)

    dump = jnp.full((16,), DUMP, jnp.int32)

    @pl.loop(0, 4 * CAP // 16)
    def _(i):
        srcb[pl.ds(i * 16, 16)] = dump
        dstb[pl.ds(i * 16, 16)] = dump

    def body(j, pos):
        for k in range(8):
            s16 = src_in[j, pl.ds(k * 16, 16)]
            d16 = dst_in[j, pl.ds(k * 16, 16)]
            gs = s16 >= HALF
            gd = d16 >= HALF
            valid = s16 != PAD_IDX
            sm = s16 - jnp.where(gs, HALF, 0)
            dm = d16 - jnp.where(gd, HALF, 0)
            for g in range(4):
                mg = (gs == (g >= 2)) & (gd == (g % 2 == 1)) & valid
                mi = jnp.where(mg, 1, 0)
                ranks = plsc.cumsum(mi)
                lidx = jnp.where(mg, g * CAP + pos[g] + ranks - 1, 4 * CAP)
                plsc.store_scatter(srcb, [lidx], sm)
                plsc.store_scatter(dstb, [lidx], dm)
                cnt = jnp.sum(mi)
                pos = tuple(p + cnt if gg == g else p
                            for gg, p in enumerate(pos))
        return pos

    lax.fori_loop(0, EPT // 128, body,
                  (jnp.int32(0), jnp.int32(0), jnp.int32(0), jnp.int32(0)))

    for g in range(4):
        pltpu.sync_copy(srcb.at[pl.ds(g * CAP, CAP)],
                        srcl_hbm.at[pl.ds((wid * 4 + g) * CAP, CAP)])
        pltpu.sync_copy(dstb.at[pl.ds(g * CAP, CAP)],
                        dstl_hbm.at[pl.ds((wid * 4 + g) * CAP, CAP)])


@functools.partial(
    pl.kernel,
    out_type=jax.ShapeDtypeStruct((2, HROWS, D), jnp.float32),
    mesh=_sc_mesh,
    scratch_types=[
        pltpu.VMEM((LCH, CHUNK), jnp.int32),
        pltpu.VMEM((LCH, CHUNK), jnp.int32),
    ] + [pltpu.VMEM((CHUNK, D), jnp.float32)] * NBUF
      + [pltpu.VMEM_SHARED((HROWS, D), jnp.float32),
         pltpu.VMEM_SHARED((HROWS, D), jnp.float32)]
      + [pltpu.SemaphoreType.DMA] * NBUF,
)
def _sc_agg(neigh_hbm, srcl_hbm, dstl_hbm, zeros_hbm, out_hbm,
            src_v, dst_v, *rest):
    bufs = rest[:NBUF]
    tab_sh = rest[NBUF]
    acc_sh = rest[NBUF + 1]
    sems = rest[NBUF + 2:]
    cid = lax.axis_index("c")
    sid = lax.axis_index("s")
    row0 = sid * ACC_RPT
    pltpu.sync_copy(zeros_hbm.at[pl.ds(row0, ACC_RPT)],
                    acc_sh.at[pl.ds(row0, ACC_RPT)])

    @pl.when(sid == 15)
    def _():
        pltpu.sync_copy(zeros_hbm.at[pl.ds(HALF, 8)],
                        acc_sh.at[pl.ds(HALF, 8)])

    def g_start(j, k):
        pltpu.async_copy(tab_sh.at[src_v.at[j]], bufs[k], sems[k])

    def g_wait(k):
        pltpu.make_async_copy(tab_sh.at[src_v.at[0]], bufs[k], sems[k]).wait()

    def s_add(j, k):
        pltpu.sync_copy(bufs[k], acc_sh.at[dst_v.at[j]], add=True)

    for p in range(2):
        pltpu.sync_copy(neigh_hbm.at[pl.ds(p * HALF + row0, ACC_RPT)],
                        tab_sh.at[pl.ds(row0, ACC_RPT)])

        @pl.when(sid == 15)
        def _():
            pltpu.sync_copy(zeros_hbm.at[pl.ds(HALF, 8)],
                            tab_sh.at[pl.ds(HALF, 8)])

        plsc.subcore_barrier()

        g = p * 2 + cid
        for oo in range(2):
            o = sid * 2 + oo
            pltpu.sync_copy(srcl_hbm.at[o, g], src_v)
            pltpu.sync_copy(dstl_hbm.at[o, g], dst_v)
            for k in range(NBUF - 1):
                g_start(k, k)

            @pl.loop(0, LCH, step=NBUF)
            def _(j):
                for k in range(NBUF):
                    nxt = j + k + NBUF - 1

                    @pl.when(nxt < LCH)
                    def _():
                        g_start(nxt, (k + NBUF - 1) % NBUF)

                    @pl.when(j + k < LCH)
                    def _():
                        g_wait(k)
                        s_add(j + k, k)

        plsc.subcore_barrier()

    pltpu.sync_copy(acc_sh.at[pl.ds(row0, ACC_RPT)],
                    out_hbm.at[cid, pl.ds(row0, ACC_RPT)])


def kernel(x, edges, W0, W1, b):
    e = edges.astype(jnp.int32)
    src = jnp.concatenate([e[:, 0], e[:, 1]])
    dst = jnp.concatenate([e[:, 1], e[:, 0]])
    pad = jnp.full((E2PAD - E2,), PAD_IDX, jnp.int32)
    srcr = jnp.concatenate([src, pad]).reshape(NW, EPT // 128, 128)
    dstr = jnp.concatenate([dst, pad]).reshape(NW, EPT // 128, 128)
    zeros = jnp.zeros((HROWS, D), jnp.float32)

    srcl, dstl = _sc_part(srcr, dstr)
    srcl = srcl.reshape(NW, 4, LCH, 128)
    dstl = dstl.reshape(NW, 4, LCH, 128)

    h = jnp.zeros((NPAD, D), jnp.float32).at[:N].set(x)
    selfp, neigh = _tc_mm(h, W0[0], W1[0], b[0].reshape(1, D))
    parts = _sc_agg(neigh, srcl, dstl, zeros)
    for l in range(1, NUM_LAYERS):
        selfp, neigh = _tc_mm_fused(selfp, parts, W0[l], W1[l],
                                    b[l].reshape(1, D))
        parts = _sc_agg(neigh, srcl, dstl, zeros)
    h = _tc_combine(selfp, parts, relu=False)
    return h[:N]
